# Initial kernel scaffold; baseline (speedup 1.0000x reference)
import jax
import jax.numpy as jnp
from jax.experimental import pallas as pl


def kernel(inputs, paths, index, sequences, features, flow_size, kernel, recurrent_kernel, bias):
    raise NotImplementedError("write your pallas kernel here")


# speedup vs baseline: 15.0334x; 15.0334x over previous
import functools

import jax
import jax.numpy as jnp
from jax import lax
from jax.experimental import pallas as pl
from jax.experimental.pallas import tpu as pltpu
from jax.experimental.pallas import tpu_sc as plsc

NUM_QUESTS = 512
NUM_PATHS = 4
LINK_DIM = 256
NUM_LINKS = 10000
MAX_LEN = 32
B = NUM_QUESTS * NUM_PATHS

NW = 32
BPW = B // NW
CHUNK = BPW * MAX_LEN + 32


def _proj_body(x_ref, w_ref, p_ref, ss_ref):
    i = pl.program_id(0)
    x = x_ref[...]
    p_ref[...] = jnp.dot(x, w_ref[...], preferred_element_type=jnp.float32)
    blk = jnp.sum(x * x)

    @pl.when(i == 0)
    def _():
        ss_ref[0, 0] = blk

    @pl.when(i > 0)
    def _():
        ss_ref[0, 0] += blk


def _project(inputs, w_pad):
    n_blk = 10
    rows = NUM_LINKS // n_blk
    return pl.pallas_call(
        _proj_body,
        grid=(n_blk,),
        in_specs=[
            pl.BlockSpec((rows, LINK_DIM), lambda i: (i, 0)),
            pl.BlockSpec((LINK_DIM, 4), lambda i: (0, 0)),
        ],
        out_specs=[
            pl.BlockSpec((rows, 4), lambda i: (i, 0)),
            pl.BlockSpec((1, 1), lambda i: (0, 0), memory_space=pltpu.SMEM),
        ],
        out_shape=[
            jax.ShapeDtypeStruct((NUM_LINKS, 4), jnp.float32),
            jax.ShapeDtypeStruct((1, 1), jnp.float32),
        ],
    )(inputs, w_pad)


def _make_scatter(total_pad):
    mesh = plsc.VectorSubcoreMesh(core_axis_name="c", subcore_axis_name="s")

    @functools.partial(
        pl.kernel,
        mesh=mesh,
        compiler_params=pltpu.CompilerParams(needs_layout_passes=False),
        out_type=[
            jax.ShapeDtypeStruct((B, MAX_LEN), jnp.float32),
            jax.ShapeDtypeStruct((B, MAX_LEN), jnp.float32),
            jax.ShapeDtypeStruct((B, MAX_LEN), jnp.float32),
            jax.ShapeDtypeStruct((B,), jnp.int32),
        ],
        scratch_types=[
            pltpu.VMEM((total_pad,), jnp.int32),
            pltpu.VMEM((4 * NUM_LINKS,), jnp.float32),
            pltpu.VMEM((CHUNK,), jnp.int32),
            pltpu.VMEM((CHUNK,), jnp.int32),
            pltpu.VMEM((BPW, MAX_LEN), jnp.float32),
            pltpu.VMEM((BPW, MAX_LEN), jnp.float32),
            pltpu.VMEM((BPW, MAX_LEN), jnp.float32),
            pltpu.VMEM((BPW,), jnp.int32),
        ],
    )
    def scatter_kernel(p_hbm, idx_hbm, seq_hbm, path_hbm,
                       oz, orr, oh, olens,
                       idx_v, p_v, path_v, seq_v, sz, sr, sh, slens):
        wid = lax.axis_index("s") * 2 + lax.axis_index("c")
        b0 = wid * BPW

        pltpu.sync_copy(idx_hbm.at[pl.ds(0, total_pad)], idx_v)
        pltpu.sync_copy(p_hbm, p_v)

        n_search = max(1, (total_pad + 1).bit_length())

        def lower_bound(target):
            def body(_, st):
                lo, hi = st
                mid = (lo + hi) // 2
                probe = jnp.full((16,), mid, jnp.int32)
                v = jnp.max(plsc.load_gather(idx_v, [probe]))
                go_right = v < target
                return (jnp.where(go_right, mid + 1, lo),
                        jnp.where(go_right, hi, mid))

            lo, _ = lax.fori_loop(0, n_search, body,
                                  (jnp.int32(0), jnp.int32(total_pad)))
            return lo

        lo_w = lower_bound(b0)
        hi_w = lower_bound(b0 + BPW)
        lo_a = (lo_w // 16) * 16

        pltpu.sync_copy(path_hbm.at[pl.ds(lo_a, CHUNK)], path_v)
        pltpu.sync_copy(seq_hbm.at[pl.ds(lo_a, CHUNK)], seq_v)

        zeros16 = jnp.zeros((16,), jnp.int32)
        for j in range(BPW // 16):
            slens[pl.ds(j * 16, 16)] = zeros16

        lane = lax.iota(jnp.int32, 16)
        ones16 = jnp.ones((16,), jnp.int32)
        niter = (hi_w - lo_a + 15) // 16

        def body(i, carry):
            off = i * 16
            k = lo_a + off
            idxv = idx_v[pl.ds(k, 16)]
            pathv = path_v[pl.ds(off, 16)]
            seqv = seq_v[pl.ds(off, 16)]
            kk = k + lane
            m = (kk >= lo_w) & (kk < hi_w)
            bl = jnp.where(m, idxv - b0, 0)
            tt = jnp.where(m, seqv, 0)
            pb = pathv * 4
            vz = plsc.load_gather(p_v, [pb])
            vr = plsc.load_gather(p_v, [pb + 1])
            vh = plsc.load_gather(p_v, [pb + 2])
            plsc.store_scatter(sz, [bl, tt], vz, mask=m)
            plsc.store_scatter(sr, [bl, tt], vr, mask=m)
            plsc.store_scatter(sh, [bl, tt], vh, mask=m)
            plsc.addupdate_scatter(slens, [bl], ones16, mask=m)
            return carry

        lax.fori_loop(0, niter, body, jnp.int32(0))

        pltpu.sync_copy(slens, olens.at[pl.ds(b0, BPW)])
        pltpu.sync_copy(sz, oz.at[pl.ds(b0, BPW), :])
        pltpu.sync_copy(sr, orr.at[pl.ds(b0, BPW), :])
        pltpu.sync_copy(sh, oh.at[pl.ds(b0, BPW), :])

    return scatter_kernel


def _gru_body(ss_ref, rk_ref, b_ref, mz_ref, mr_ref, mh_ref, lens_ref, h0_ref,
              out_ref):
    s = lax.rsqrt(jnp.maximum(ss_ref[0, 0], 1e-12))
    rk0 = rk_ref[0, 0]
    rk1 = rk_ref[0, 1]
    rk2 = rk_ref[0, 2]
    b00 = b_ref[0, 0]
    b01 = b_ref[0, 1]
    b02 = b_ref[0, 2]
    b10 = b_ref[1, 0]
    b11 = b_ref[1, 1]
    b12 = b_ref[1, 2]
    h = h0_ref[...]
    lens = lens_ref[...]
    for t in range(MAX_LEN):
        xz = mz_ref[t] * s + b00
        xr = mr_ref[t] * s + b01
        xh = mh_ref[t] * s + b02
        z = jax.nn.sigmoid(xz + h * rk0 + b10)
        r = jax.nn.sigmoid(xr + h * rk1 + b11)
        hh = jnp.tanh(xh + r * (h * rk2 + b12))
        h = jnp.where(t < lens, z * h + (1.0 - z) * hh, h)
    out_ref[...] = h


def _gru_scan(ss, rk, bias, mz, mr, mh, lens, h0):
    smem = pl.BlockSpec(memory_space=pltpu.SMEM)
    vmem = pl.BlockSpec(memory_space=pltpu.VMEM)
    return pl.pallas_call(
        _gru_body,
        in_specs=[smem, smem, smem, vmem, vmem, vmem, vmem, vmem],
        out_specs=pl.BlockSpec(memory_space=pltpu.VMEM),
        out_shape=jax.ShapeDtypeStruct((16, 128), jnp.float32),
    )(ss, rk, bias, mz, mr, mh, lens, h0)


def kernel(inputs, paths, index, sequences, features, flow_size,
           kernel, recurrent_kernel, bias):
    del features
    w_pad = jnp.pad(kernel, ((0, 0), (0, 1)))
    p, ss = _project(inputs, w_pad)

    total = paths.shape[0]
    total_pad = total + (-total) % 16
    pad = total_pad + CHUNK - total
    idx_p = jnp.pad(index, (0, pad), constant_values=B)
    seq_p = jnp.pad(sequences, (0, pad))
    path_p = jnp.pad(paths, (0, pad))

    mz, mr, mh, lens = _make_scatter(total_pad)(
        p.reshape(-1), idx_p, seq_p, path_p)

    out = _gru_scan(ss, recurrent_kernel, bias,
                    mz.T.reshape(MAX_LEN, 16, 128),
                    mr.T.reshape(MAX_LEN, 16, 128),
                    mh.T.reshape(MAX_LEN, 16, 128),
                    lens.reshape(16, 128),
                    flow_size.reshape(16, 128))
    return out.reshape(NUM_QUESTS, NUM_PATHS)
